# baseline (speedup 1.0000x reference)
import functools

import jax
import jax.numpy as jnp
from jax import lax
from jax.experimental import pallas as pl
from jax.experimental.pallas import tpu as pltpu
from jax.experimental.pallas import tpu_sc as plsc

D = 128
_NC = 2
_NS = 16
_NW = _NC * _NS
_K = 128
_W = 32
_NCH = D // _W


def _pad_to(n, m):
    return ((n + m - 1) // m) * m


def _rows_grid(n, br):
    return (n + br - 1) // br


def _dot(a, b):
    return jnp.dot(a, b, preferred_element_type=jnp.float32)


_RMAX = 9984


def _seg_ranges(ndp):
    np_ = (ndp + _RMAX - 1) // _RMAX
    los = [p * _RMAX for p in range(np_)]
    lens = [min(_RMAX, ndp - lo) for lo in los]
    return los, lens


def _sc_segsum(x, src, dst, ndp):
    e_pad = src.shape[0]
    per_tile = e_pad // _NW
    nb = per_tile // _K
    assert nb % 2 == 0
    los, lens = _seg_ranges(ndp)
    R = lens[0]
    mesh = plsc.VectorSubcoreMesh(core_axis_name="c", subcore_axis_name="s")

    @functools.partial(
        pl.kernel, mesh=mesh,
        out_type=jax.ShapeDtypeStruct((_NC, ndp, D), jnp.float32),
        scratch_types=[
            pltpu.VMEM((_K,), jnp.int32),
            pltpu.VMEM((_K,), jnp.int32),
            pltpu.VMEM((_K,), jnp.int32),
            pltpu.VMEM((_K,), jnp.int32),
            pltpu.VMEM((_K, D), jnp.float32),
            pltpu.VMEM((_K, D), jnp.float32),
            pltpu.VMEM((128, D), jnp.float32),
            pltpu.VMEM_SHARED((R + 16, D), jnp.float32),
            pltpu.SemaphoreType.DMA,
            pltpu.SemaphoreType.DMA,
        ],
    )
    def k(x_hbm, src_hbm, dst_hbm, out_hbm,
          src_a, dst_a, src_b, dst_b, rows_a, rows_b, zer_v, acc_sh,
          sem_a, sem_b):
        cid = lax.axis_index("c")
        sid = lax.axis_index("s")
        wid = sid * _NC + cid
        z16 = jnp.zeros((16,), jnp.float32)

        def zb(i, _):
            for j in range(D // 16):
                zer_v[i, pl.ds(j * 16, 16)] = z16
            return 0
        lax.fori_loop(0, 128, zb, 0)

        for p, (lo, rl) in enumerate(zip(los, lens)):
            assert rl % _NS == 0
            rpt = rl // _NS
            r0 = sid * rpt
            off = 0
            while off < rpt:
                step = min(128, rpt - off)
                pltpu.sync_copy(zer_v.at[pl.ds(0, step)],
                                acc_sh.at[pl.ds(r0 + off, step)])
                off += step
            plsc.subcore_barrier()

            def prefetch(b, src_v, dst_v, sem):
                base = wid * per_tile + b * _K
                pltpu.sync_copy(src_hbm.at[pl.ds(base, _K)], src_v)
                pltpu.sync_copy(dst_hbm.at[pl.ds(base, _K)], dst_v)

                def adj(j, _):
                    d16 = dst_v[pl.ds(j * 16, 16)]
                    s16 = src_v[pl.ds(j * 16, 16)]
                    inr = (d16 >= lo) & (d16 < lo + rl)
                    dst_v[pl.ds(j * 16, 16)] = jnp.where(
                        inr, d16 - lo, R + (d16 & 15))
                    src_v[pl.ds(j * 16, 16)] = jnp.where(inr, s16, 0)
                    return 0
                lax.fori_loop(0, _K // 16, adj, 0)

            def start(src_v, rows_v, sem):
                pltpu.async_copy(x_hbm.at[src_v], rows_v, sem)

            def finish(src_v, dst_v, rows_v, sem):
                pltpu.make_async_copy(
                    x_hbm.at[src_v], rows_v, sem).wait()
                pltpu.sync_copy(rows_v, acc_sh.at[dst_v], add=True)

            prefetch(0, src_a, dst_a, sem_a)
            start(src_a, rows_a, sem_a)

            def pair(q, _):
                b0 = 2 * q
                prefetch(b0 + 1, src_b, dst_b, sem_b)
                start(src_b, rows_b, sem_b)
                finish(src_a, dst_a, rows_a, sem_a)
                prefetch(b0 + 2, src_a, dst_a, sem_a)
                start(src_a, rows_a, sem_a)
                finish(src_b, dst_b, rows_b, sem_b)
                return 0
            lax.fori_loop(0, nb // 2 - 1, pair, 0)

            prefetch(nb - 1, src_b, dst_b, sem_b)
            start(src_b, rows_b, sem_b)
            finish(src_a, dst_a, rows_a, sem_a)
            finish(src_b, dst_b, rows_b, sem_b)

            plsc.subcore_barrier()
            pltpu.sync_copy(acc_sh.at[pl.ds(r0, rpt)],
                            out_hbm.at[cid, pl.ds(lo + r0, rpt)])
            plsc.subcore_barrier()

    return k(x, src, dst)


def _sc_count(dst, ndp):
    e_pad = dst.shape[0]
    per_tile = e_pad // _NW
    nb = per_tile // _K
    assert nb % 2 == 0
    los, lens = _seg_ranges(ndp)
    R = lens[0]
    mesh = plsc.VectorSubcoreMesh(core_axis_name="c", subcore_axis_name="s")

    @functools.partial(
        pl.kernel, mesh=mesh,
        out_type=jax.ShapeDtypeStruct((_NC, ndp, D), jnp.float32),
        scratch_types=[
            pltpu.VMEM((_K,), jnp.int32),
            pltpu.VMEM((_K,), jnp.int32),
            pltpu.VMEM((_K, D), jnp.float32),
            pltpu.VMEM((128, D), jnp.float32),
            pltpu.VMEM_SHARED((R + 16, D), jnp.float32),
        ],
    )
    def k(dst_hbm, out_hbm, dst_a, dst_b, ones_v, zer_v, acc_sh):
        cid = lax.axis_index("c")
        sid = lax.axis_index("s")
        wid = sid * _NC + cid
        z16 = jnp.zeros((16,), jnp.float32)
        o16 = jnp.ones((16,), jnp.float32)

        def ib(i, _):
            for j in range(D // 16):
                ones_v[i, pl.ds(j * 16, 16)] = o16
            return 0
        lax.fori_loop(0, _K, ib, 0)

        def zb(i, _):
            for j in range(D // 16):
                zer_v[i, pl.ds(j * 16, 16)] = z16
            return 0
        lax.fori_loop(0, 128, zb, 0)

        for p, (lo, rl) in enumerate(zip(los, lens)):
            assert rl % _NS == 0
            rpt = rl // _NS
            r0 = sid * rpt
            off = 0
            while off < rpt:
                step = min(128, rpt - off)
                pltpu.sync_copy(zer_v.at[pl.ds(0, step)],
                                acc_sh.at[pl.ds(r0 + off, step)])
                off += step
            plsc.subcore_barrier()

            def prefetch(b, dst_v):
                base = wid * per_tile + b * _K
                pltpu.sync_copy(dst_hbm.at[pl.ds(base, _K)], dst_v)

                def adj(j, _):
                    d16 = dst_v[pl.ds(j * 16, 16)]
                    inr = (d16 >= lo) & (d16 < lo + rl)
                    dst_v[pl.ds(j * 16, 16)] = jnp.where(
                        inr, d16 - lo, R + (d16 & 15))
                    return 0
                lax.fori_loop(0, _K // 16, adj, 0)

            def scatter(dst_v):
                pltpu.sync_copy(ones_v, acc_sh.at[dst_v], add=True)

            prefetch(0, dst_a)

            def pair(q, _):
                b0 = 2 * q
                prefetch(b0 + 1, dst_b)
                scatter(dst_a)
                prefetch(b0 + 2, dst_a)
                scatter(dst_b)
                return 0
            lax.fori_loop(0, nb // 2 - 1, pair, 0)

            prefetch(nb - 1, dst_b)
            scatter(dst_a)
            scatter(dst_b)

            plsc.subcore_barrier()
            pltpu.sync_copy(acc_sh.at[pl.ds(r0, rpt)],
                            out_hbm.at[cid, pl.ds(lo + r0, rpt)])
            plsc.subcore_barrier()

    return k(dst)


def _sc_gather(h, idx):
    e_pad = idx.shape[0]
    per_tile = e_pad // _NW
    nb = per_tile // _K
    mesh = plsc.VectorSubcoreMesh(core_axis_name="c", subcore_axis_name="s")

    assert nb % 2 == 0

    @functools.partial(
        pl.kernel, mesh=mesh,
        out_type=jax.ShapeDtypeStruct((e_pad, D), jnp.float32),
        scratch_types=[
            pltpu.VMEM((_K,), jnp.int32),
            pltpu.VMEM((_K,), jnp.int32),
            pltpu.VMEM((_K, D), jnp.float32),
            pltpu.VMEM((_K, D), jnp.float32),
            pltpu.SemaphoreType.DMA,
            pltpu.SemaphoreType.DMA,
        ],
    )
    def k(h_hbm, idx_hbm, out_hbm, idx_a, idx_b, rows_a, rows_b,
          sem_a, sem_b):
        cid = lax.axis_index("c")
        sid = lax.axis_index("s")
        wid = sid * _NC + cid

        def start(b, idx_v, rows_v, sem):
            base = wid * per_tile + b * _K
            pltpu.sync_copy(idx_hbm.at[pl.ds(base, _K)], idx_v)
            pltpu.async_copy(h_hbm.at[idx_v], rows_v, sem)

        def finish(b, idx_v, rows_v, sem):
            base = wid * per_tile + b * _K
            pltpu.make_async_copy(h_hbm.at[idx_v], rows_v, sem).wait()
            pltpu.sync_copy(rows_v, out_hbm.at[pl.ds(base, _K)])

        start(0, idx_a, rows_a, sem_a)

        def pair(q, _):
            b0 = 2 * q
            start(b0 + 1, idx_b, rows_b, sem_b)
            finish(b0, idx_a, rows_a, sem_a)
            start(b0 + 2, idx_a, rows_a, sem_a)
            finish(b0 + 1, idx_b, rows_b, sem_b)
            return 0
        lax.fori_loop(0, nb // 2 - 1, pair, 0)

        start(nb - 1, idx_b, rows_b, sem_b)
        finish(nb - 2, idx_a, rows_a, sem_a)
        finish(nb - 1, idx_b, rows_b, sem_b)

    return k(h, idx)


def _proj_body(x_ref, w_ref, b_ref, o_ref):
    o_ref[...] = _dot(x_ref[...], w_ref[...]) + b_ref[...]


def _proj(x, W, b, br=512):
    n = x.shape[0]
    return pl.pallas_call(
        _proj_body,
        grid=(_rows_grid(n, br),),
        in_specs=[
            pl.BlockSpec((br, D), lambda i: (i, 0)),
            pl.BlockSpec((D, D), lambda i: (0, 0)),
            pl.BlockSpec((1, D), lambda i: (0, 0)),
        ],
        out_specs=pl.BlockSpec((br, D), lambda i: (i, 0)),
        out_shape=jax.ShapeDtypeStruct((n, D), jnp.float32),
    )(x, W, b.reshape(1, D))


def _rel_out(s_ref, c_ref, wl_ref, bl_ref, x_ref, wr_ref):
    c = jnp.maximum(c_ref[0, :, 0:1] + c_ref[1, :, 0:1], 1.0)
    m = (s_ref[0] + s_ref[1]) / c
    return (_dot(m, wl_ref[...]) + bl_ref[...]) + _dot(x_ref[...], wr_ref[...])


def _comb2_body(s0_ref, c0_ref, wl0_ref, bl0_ref, wr0_ref,
                s1_ref, c1_ref, wl1_ref, bl1_ref, wr1_ref,
                x_ref, o_ref, *, scale):
    o0 = _rel_out(s0_ref, c0_ref, wl0_ref, bl0_ref, x_ref, wr0_ref)
    o1 = _rel_out(s1_ref, c1_ref, wl1_ref, bl1_ref, x_ref, wr1_ref)
    o_ref[...] = (o0 + o1) * scale


def _comb1_body(s0_ref, c0_ref, wl0_ref, bl0_ref, wr0_ref,
                x_ref, o_ref, *, scale):
    o_ref[...] = _rel_out(s0_ref, c0_ref, wl0_ref, bl0_ref,
                          x_ref, wr0_ref) * scale


def _combine(sums, cnts, wls, bls, wrs, x, br=512):
    n = x.shape[0]
    ndp = sums[0].shape[1]
    nrel = len(sums)
    scale = 1.0 / nrel
    row = lambda i: (i, 0)
    sspec = pl.BlockSpec((_NC, br, D), lambda i: (0, i, 0))
    cspec = pl.BlockSpec((_NC, br, 16), lambda i: (0, i, 0))
    mat = pl.BlockSpec((br, D), row)
    wspec = pl.BlockSpec((D, D), lambda i: (0, 0))
    bspec = pl.BlockSpec((1, D), lambda i: (0, 0))
    assert ndp % br == 0 and ndp >= n
    if nrel == 2:
        body = functools.partial(_comb2_body, scale=scale)
        in_specs = [sspec, cspec, wspec, bspec, wspec,
                    sspec, cspec, wspec, bspec, wspec, mat]
        args = (sums[0], cnts[0], wls[0], bls[0].reshape(1, D), wrs[0],
                sums[1], cnts[1], wls[1], bls[1].reshape(1, D), wrs[1], x)
    else:
        body = functools.partial(_comb1_body, scale=scale)
        in_specs = [sspec, cspec, wspec, bspec, wspec, mat]
        args = (sums[0], cnts[0], wls[0], bls[0].reshape(1, D), wrs[0], x)
    return pl.pallas_call(
        body,
        grid=(_rows_grid(n, br),),
        in_specs=in_specs,
        out_specs=pl.BlockSpec((br, D), row),
        out_shape=jax.ShapeDtypeStruct((n, D), jnp.float32),
    )(*args)


def _mlp_body(z0_ref, z1_ref, w0a_ref, w0b_ref, b0_ref, w1_ref, b1_ref,
              w2_ref, b2_ref, w3_ref, b3_ref, w4_ref, b4_ref, o_ref):
    h = _dot(z0_ref[...], w0a_ref[...])
    h += _dot(z1_ref[...], w0b_ref[...])
    h = jax.nn.relu(h + b0_ref[...])
    h = jax.nn.relu(_dot(h, w1_ref[...]) + b1_ref[...])
    h = jax.nn.relu(_dot(h, w2_ref[...]) + b2_ref[...])
    h = jax.nn.relu(_dot(h, w3_ref[...]) + b3_ref[...])
    o_ref[...] = _dot(h, w4_ref[...]) + b4_ref[...]


def _mlp(z0, z1, W0, b0, W1, b1, W2, b2, W3, b3, W4, b4, br=512):
    n = z0.shape[0]
    fix = lambda i: (0, 0)
    row = lambda i: (i, 0)
    W4p = jnp.pad(W4, ((0, 0), (0, 7)))
    b4p = jnp.pad(b4, (0, 7))
    out = pl.pallas_call(
        _mlp_body,
        grid=(_rows_grid(n, br),),
        in_specs=[
            pl.BlockSpec((br, D), row),
            pl.BlockSpec((br, D), row),
            pl.BlockSpec((D, 512), fix),
            pl.BlockSpec((D, 512), fix),
            pl.BlockSpec((1, 512), fix),
            pl.BlockSpec((512, 256), fix),
            pl.BlockSpec((1, 256), fix),
            pl.BlockSpec((256, 128), fix),
            pl.BlockSpec((1, 128), fix),
            pl.BlockSpec((128, 64), fix),
            pl.BlockSpec((1, 64), fix),
            pl.BlockSpec((64, 8), fix),
            pl.BlockSpec((1, 8), fix),
        ],
        out_specs=pl.BlockSpec((br, 8), row),
        out_shape=jax.ShapeDtypeStruct((n, 8), jnp.float32),
    )(z0, z1, W0[:D], W0[D:], b0.reshape(1, 512), W1, b1.reshape(1, 256),
      W2, b2.reshape(1, 128), W3, b3.reshape(1, 64), W4p, b4p.reshape(1, 8))
    return out[:, 0]


_RELS = [('before_ep', 'OER', 'OER'), ('covers', 'OER', 'Concept'),
         ('belongs', 'Concept', 'Class'), ('rev_covers', 'Concept', 'OER'),
         ('rev_belongs', 'Class', 'Concept')]


def _pad_idx(v, fill):
    e = v.shape[0]
    e_pad = _pad_to(e, _NW * _K * 2)
    if e_pad == e:
        return v
    return jnp.concatenate(
        [v, jnp.full((e_pad - e,), fill, jnp.int32)])


def kernel(x_oer, x_concept, x_class, edge_label_index_before_sr, edge_index_before_ep, edge_index_covers, edge_index_belongs, edge_index_rev_covers, edge_index_rev_belongs, lin_oer_W, lin_oer_b, lin_concept_W, lin_concept_b, lin_class_W, lin_class_b, l0_before_ep_Wl, l0_before_ep_bl, l0_before_ep_Wr, l0_covers_Wl, l0_covers_bl, l0_covers_Wr, l0_belongs_Wl, l0_belongs_bl, l0_belongs_Wr, l0_rev_covers_Wl, l0_rev_covers_bl, l0_rev_covers_Wr, l0_rev_belongs_Wl, l0_rev_belongs_bl, l0_rev_belongs_Wr, l1_before_ep_Wl, l1_before_ep_bl, l1_before_ep_Wr, l1_covers_Wl, l1_covers_bl, l1_covers_Wr, l1_belongs_Wl, l1_belongs_bl, l1_belongs_Wr, l1_rev_covers_Wl, l1_rev_covers_bl, l1_rev_covers_Wr, l1_rev_belongs_Wl, l1_rev_belongs_bl, l1_rev_belongs_Wr, mlp_W0, mlp_b0, mlp_W1, mlp_b1, mlp_W2, mlp_b2, mlp_W3, mlp_b3, mlp_W4, mlp_b4):
    params = dict(locals())
    x = {
        'OER': _proj(x_oer, lin_oer_W, lin_oer_b),
        'Concept': _proj(x_concept, lin_concept_W, lin_concept_b),
        'Class': _proj(x_class, lin_class_W, lin_class_b),
    }
    nn = {t: x[t].shape[0] for t in x}
    ndp = {t: _pad_to(nn[t] + 1, 512) for t in x}
    edges = {
        'before_ep': edge_index_before_ep,
        'covers': edge_index_covers,
        'belongs': edge_index_belongs,
        'rev_covers': edge_index_rev_covers,
        'rev_belongs': edge_index_rev_belongs,
    }
    epad = {}
    cnt = {}
    for rel, src_t, dst_t in _RELS:
        ei = edges[rel]
        src = _pad_idx(ei[0], 0)
        dst = _pad_idx(ei[1], nn[dst_t])
        epad[rel] = (src, dst)
        cnt[rel] = _sc_count(dst, ndp[dst_t])[:, :, :16]

    for l in range(2):
        acc = {'OER': [], 'Concept': [], 'Class': []}
        for rel, src_t, dst_t in _RELS:
            src, dst = epad[rel]
            s = _sc_segsum(x[src_t], src, dst, ndp[dst_t])
            acc[dst_t].append((s, cnt[rel],
                               params['l%d_%s_Wl' % (l, rel)],
                               params['l%d_%s_bl' % (l, rel)],
                               params['l%d_%s_Wr' % (l, rel)]))
        newx = {}
        for t, items in acc.items():
            newx[t] = _combine([it[0] for it in items],
                               [it[1] for it in items],
                               [it[2] for it in items],
                               [it[3] for it in items],
                               [it[4] for it in items], x[t])
        x = newx

    e = edge_label_index_before_sr
    h = x['OER']
    z0 = _sc_gather(h, _pad_idx(e[0], 0))
    z1 = _sc_gather(h, _pad_idx(e[1], 0))
    n_lab = e.shape[1]
    out = _mlp(z0, z1, mlp_W0, mlp_b0, mlp_W1, mlp_b1, mlp_W2, mlp_b2,
               mlp_W3, mlp_b3, mlp_W4, mlp_b4)
    return out[:n_lab]


# speedup vs baseline: 23.1331x; 23.1331x over previous
import functools

import jax
import jax.numpy as jnp
from jax import lax
from jax.experimental import pallas as pl
from jax.experimental.pallas import tpu as pltpu
from jax.experimental.pallas import tpu_sc as plsc

D = 128
_NC = 2
_NS = 16
_NW = _NC * _NS
_K = 128
_W = 32
_NCH = D // _W


def _pad_to(n, m):
    return ((n + m - 1) // m) * m


def _rows_grid(n, br):
    return (n + br - 1) // br


def _dot(a, b):
    return jnp.dot(a, b, preferred_element_type=jnp.float32)


_RMAX = 9984


def _seg_ranges(ndp):
    np_ = (ndp + _RMAX - 1) // _RMAX
    los = [p * _RMAX for p in range(np_)]
    lens = [min(_RMAX, ndp - lo) for lo in los]
    return los, lens


def _sc_segsum(x, src, dst, ndp):
    e_pad = src.shape[0]
    per_tile = e_pad // _NW
    nb = per_tile // _K
    assert nb % 2 == 0
    los, lens = _seg_ranges(ndp)
    R = lens[0]
    mesh = plsc.VectorSubcoreMesh(core_axis_name="c", subcore_axis_name="s")

    @functools.partial(
        pl.kernel, mesh=mesh,
        out_type=jax.ShapeDtypeStruct((_NC, ndp, D), jnp.float32),
        scratch_types=[
            pltpu.VMEM((_K,), jnp.int32),
            pltpu.VMEM((_K,), jnp.int32),
            pltpu.VMEM((_K,), jnp.int32),
            pltpu.VMEM((_K,), jnp.int32),
            pltpu.VMEM((_K, D), jnp.float32),
            pltpu.VMEM((_K, D), jnp.float32),
            pltpu.VMEM((128, D), jnp.float32),
            pltpu.VMEM_SHARED((R + 16, D), jnp.float32),
            pltpu.SemaphoreType.DMA,
            pltpu.SemaphoreType.DMA,
        ],
    )
    def k(x_hbm, src_hbm, dst_hbm, out_hbm,
          src_a, dst_a, src_b, dst_b, rows_a, rows_b, zer_v, acc_sh,
          sem_a, sem_b):
        cid = lax.axis_index("c")
        sid = lax.axis_index("s")
        wid = sid * _NC + cid
        z16 = jnp.zeros((16,), jnp.float32)

        def zb(i, _):
            for j in range(D // 16):
                zer_v[i, pl.ds(j * 16, 16)] = z16
            return 0
        lax.fori_loop(0, 128, zb, 0)

        for p, (lo, rl) in enumerate(zip(los, lens)):
            assert rl % _NS == 0
            rpt = rl // _NS
            r0 = sid * rpt
            off = 0
            while off < rpt:
                step = min(128, rpt - off)
                pltpu.sync_copy(zer_v.at[pl.ds(0, step)],
                                acc_sh.at[pl.ds(r0 + off, step)])
                off += step
            plsc.subcore_barrier()

            def prefetch(b, src_v, dst_v, sem):
                base = wid * per_tile + b * _K
                pltpu.sync_copy(src_hbm.at[pl.ds(base, _K)], src_v)
                pltpu.sync_copy(dst_hbm.at[pl.ds(base, _K)], dst_v)

                def adj(j, _):
                    d16 = dst_v[pl.ds(j * 16, 16)]
                    inr = (d16 >= lo) & (d16 < lo + rl)
                    dst_v[pl.ds(j * 16, 16)] = jnp.where(
                        inr, d16 - lo, R + (d16 & 15))
                    return 0
                lax.fori_loop(0, _K // 16, adj, 0)

            def start(src_v, rows_v, sem):
                pltpu.async_copy(x_hbm.at[src_v], rows_v, sem)

            def finish(src_v, dst_v, rows_v, sem):
                pltpu.make_async_copy(
                    x_hbm.at[src_v], rows_v, sem).wait()
                pltpu.sync_copy(rows_v, acc_sh.at[dst_v], add=True)

            prefetch(0, src_a, dst_a, sem_a)
            start(src_a, rows_a, sem_a)

            def pair(q, _):
                b0 = 2 * q
                prefetch(b0 + 1, src_b, dst_b, sem_b)
                start(src_b, rows_b, sem_b)
                finish(src_a, dst_a, rows_a, sem_a)
                prefetch(b0 + 2, src_a, dst_a, sem_a)
                start(src_a, rows_a, sem_a)
                finish(src_b, dst_b, rows_b, sem_b)
                return 0
            lax.fori_loop(0, nb // 2 - 1, pair, 0)

            prefetch(nb - 1, src_b, dst_b, sem_b)
            start(src_b, rows_b, sem_b)
            finish(src_a, dst_a, rows_a, sem_a)
            finish(src_b, dst_b, rows_b, sem_b)

            plsc.subcore_barrier()
            pltpu.sync_copy(acc_sh.at[pl.ds(r0, rpt)],
                            out_hbm.at[cid, pl.ds(lo + r0, rpt)])
            plsc.subcore_barrier()

    return k(x, src, dst)


def _sc_count(dst, ndp):
    e_pad = dst.shape[0]
    per_tile = e_pad // _NW
    nb = per_tile // _K
    assert nb % 2 == 0
    los, lens = _seg_ranges(ndp)
    R = lens[0]
    mesh = plsc.VectorSubcoreMesh(core_axis_name="c", subcore_axis_name="s")

    @functools.partial(
        pl.kernel, mesh=mesh,
        out_type=jax.ShapeDtypeStruct((_NC, ndp, D), jnp.float32),
        scratch_types=[
            pltpu.VMEM((_K,), jnp.int32),
            pltpu.VMEM((_K,), jnp.int32),
            pltpu.VMEM((_K, D), jnp.float32),
            pltpu.VMEM((128, D), jnp.float32),
            pltpu.VMEM_SHARED((R + 16, D), jnp.float32),
        ],
    )
    def k(dst_hbm, out_hbm, dst_a, dst_b, ones_v, zer_v, acc_sh):
        cid = lax.axis_index("c")
        sid = lax.axis_index("s")
        wid = sid * _NC + cid
        z16 = jnp.zeros((16,), jnp.float32)
        o16 = jnp.ones((16,), jnp.float32)

        def ib(i, _):
            for j in range(D // 16):
                ones_v[i, pl.ds(j * 16, 16)] = o16
            return 0
        lax.fori_loop(0, _K, ib, 0)

        def zb(i, _):
            for j in range(D // 16):
                zer_v[i, pl.ds(j * 16, 16)] = z16
            return 0
        lax.fori_loop(0, 128, zb, 0)

        for p, (lo, rl) in enumerate(zip(los, lens)):
            assert rl % _NS == 0
            rpt = rl // _NS
            r0 = sid * rpt
            off = 0
            while off < rpt:
                step = min(128, rpt - off)
                pltpu.sync_copy(zer_v.at[pl.ds(0, step)],
                                acc_sh.at[pl.ds(r0 + off, step)])
                off += step
            plsc.subcore_barrier()

            def prefetch(b, dst_v):
                base = wid * per_tile + b * _K
                pltpu.sync_copy(dst_hbm.at[pl.ds(base, _K)], dst_v)

                def adj(j, _):
                    d16 = dst_v[pl.ds(j * 16, 16)]
                    inr = (d16 >= lo) & (d16 < lo + rl)
                    dst_v[pl.ds(j * 16, 16)] = jnp.where(
                        inr, d16 - lo, R + (d16 & 15))
                    return 0
                lax.fori_loop(0, _K // 16, adj, 0)

            def scatter(dst_v):
                pltpu.sync_copy(ones_v, acc_sh.at[dst_v], add=True)

            prefetch(0, dst_a)

            def pair(q, _):
                b0 = 2 * q
                prefetch(b0 + 1, dst_b)
                scatter(dst_a)
                prefetch(b0 + 2, dst_a)
                scatter(dst_b)
                return 0
            lax.fori_loop(0, nb // 2 - 1, pair, 0)

            prefetch(nb - 1, dst_b)
            scatter(dst_a)
            scatter(dst_b)

            plsc.subcore_barrier()
            pltpu.sync_copy(acc_sh.at[pl.ds(r0, rpt)],
                            out_hbm.at[cid, pl.ds(lo + r0, rpt)])
            plsc.subcore_barrier()

    return k(dst)


def _sc_gather(h, idx):
    e_pad = idx.shape[0]
    per_tile = e_pad // _NW
    nb = per_tile // _K
    mesh = plsc.VectorSubcoreMesh(core_axis_name="c", subcore_axis_name="s")

    assert nb % 2 == 0

    @functools.partial(
        pl.kernel, mesh=mesh,
        out_type=jax.ShapeDtypeStruct((e_pad, D), jnp.float32),
        scratch_types=[
            pltpu.VMEM((_K,), jnp.int32),
            pltpu.VMEM((_K,), jnp.int32),
            pltpu.VMEM((_K, D), jnp.float32),
            pltpu.VMEM((_K, D), jnp.float32),
            pltpu.SemaphoreType.DMA,
            pltpu.SemaphoreType.DMA,
        ],
    )
    def k(h_hbm, idx_hbm, out_hbm, idx_a, idx_b, rows_a, rows_b,
          sem_a, sem_b):
        cid = lax.axis_index("c")
        sid = lax.axis_index("s")
        wid = sid * _NC + cid

        def start(b, idx_v, rows_v, sem):
            base = wid * per_tile + b * _K
            pltpu.sync_copy(idx_hbm.at[pl.ds(base, _K)], idx_v)
            pltpu.async_copy(h_hbm.at[idx_v], rows_v, sem)

        def finish(b, idx_v, rows_v, sem):
            base = wid * per_tile + b * _K
            pltpu.make_async_copy(h_hbm.at[idx_v], rows_v, sem).wait()
            pltpu.sync_copy(rows_v, out_hbm.at[pl.ds(base, _K)])

        start(0, idx_a, rows_a, sem_a)

        def pair(q, _):
            b0 = 2 * q
            start(b0 + 1, idx_b, rows_b, sem_b)
            finish(b0, idx_a, rows_a, sem_a)
            start(b0 + 2, idx_a, rows_a, sem_a)
            finish(b0 + 1, idx_b, rows_b, sem_b)
            return 0
        lax.fori_loop(0, nb // 2 - 1, pair, 0)

        start(nb - 1, idx_b, rows_b, sem_b)
        finish(nb - 2, idx_a, rows_a, sem_a)
        finish(nb - 1, idx_b, rows_b, sem_b)

    return k(h, idx)


def _proj_body(x_ref, w_ref, b_ref, o_ref):
    o_ref[...] = _dot(x_ref[...], w_ref[...]) + b_ref[...]


def _proj(x, W, b, br=512):
    n = x.shape[0]
    return pl.pallas_call(
        _proj_body,
        grid=(_rows_grid(n, br),),
        in_specs=[
            pl.BlockSpec((br, D), lambda i: (i, 0)),
            pl.BlockSpec((D, D), lambda i: (0, 0)),
            pl.BlockSpec((1, D), lambda i: (0, 0)),
        ],
        out_specs=pl.BlockSpec((br, D), lambda i: (i, 0)),
        out_shape=jax.ShapeDtypeStruct((n, D), jnp.float32),
    )(x, W, b.reshape(1, D))


def _rel_out(s_ref, c_ref, wl_ref, bl_ref, x_ref, wr_ref):
    c = jnp.maximum(c_ref[0, :, 0:1] + c_ref[1, :, 0:1], 1.0)
    m = (s_ref[0] + s_ref[1]) / c
    return (_dot(m, wl_ref[...]) + bl_ref[...]) + _dot(x_ref[...], wr_ref[...])


def _comb2_body(s0_ref, c0_ref, wl0_ref, bl0_ref, wr0_ref,
                s1_ref, c1_ref, wl1_ref, bl1_ref, wr1_ref,
                x_ref, o_ref, *, scale):
    o0 = _rel_out(s0_ref, c0_ref, wl0_ref, bl0_ref, x_ref, wr0_ref)
    o1 = _rel_out(s1_ref, c1_ref, wl1_ref, bl1_ref, x_ref, wr1_ref)
    o_ref[...] = (o0 + o1) * scale


def _comb1_body(s0_ref, c0_ref, wl0_ref, bl0_ref, wr0_ref,
                x_ref, o_ref, *, scale):
    o_ref[...] = _rel_out(s0_ref, c0_ref, wl0_ref, bl0_ref,
                          x_ref, wr0_ref) * scale


def _combine(sums, cnts, wls, bls, wrs, x, br=512):
    n = x.shape[0]
    ndp = sums[0].shape[1]
    nrel = len(sums)
    scale = 1.0 / nrel
    row = lambda i: (i, 0)
    sspec = pl.BlockSpec((_NC, br, D), lambda i: (0, i, 0))
    cspec = pl.BlockSpec((_NC, br, 16), lambda i: (0, i, 0))
    mat = pl.BlockSpec((br, D), row)
    wspec = pl.BlockSpec((D, D), lambda i: (0, 0))
    bspec = pl.BlockSpec((1, D), lambda i: (0, 0))
    assert ndp % br == 0 and ndp >= n
    if nrel == 2:
        body = functools.partial(_comb2_body, scale=scale)
        in_specs = [sspec, cspec, wspec, bspec, wspec,
                    sspec, cspec, wspec, bspec, wspec, mat]
        args = (sums[0], cnts[0], wls[0], bls[0].reshape(1, D), wrs[0],
                sums[1], cnts[1], wls[1], bls[1].reshape(1, D), wrs[1], x)
    else:
        body = functools.partial(_comb1_body, scale=scale)
        in_specs = [sspec, cspec, wspec, bspec, wspec, mat]
        args = (sums[0], cnts[0], wls[0], bls[0].reshape(1, D), wrs[0], x)
    return pl.pallas_call(
        body,
        grid=(_rows_grid(n, br),),
        in_specs=in_specs,
        out_specs=pl.BlockSpec((br, D), row),
        out_shape=jax.ShapeDtypeStruct((n, D), jnp.float32),
    )(*args)


def _mlp_body(z0_ref, z1_ref, w0a_ref, w0b_ref, b0_ref, w1_ref, b1_ref,
              w2_ref, b2_ref, w3_ref, b3_ref, w4_ref, b4_ref, o_ref):
    h = _dot(z0_ref[...], w0a_ref[...])
    h += _dot(z1_ref[...], w0b_ref[...])
    h = jax.nn.relu(h + b0_ref[...])
    h = jax.nn.relu(_dot(h, w1_ref[...]) + b1_ref[...])
    h = jax.nn.relu(_dot(h, w2_ref[...]) + b2_ref[...])
    h = jax.nn.relu(_dot(h, w3_ref[...]) + b3_ref[...])
    o_ref[...] = _dot(h, w4_ref[...]) + b4_ref[...]


def _mlp(z0, z1, W0, b0, W1, b1, W2, b2, W3, b3, W4, b4, br=512):
    n = z0.shape[0]
    fix = lambda i: (0, 0)
    row = lambda i: (i, 0)
    W4p = jnp.pad(W4, ((0, 0), (0, 7)))
    b4p = jnp.pad(b4, (0, 7))
    out = pl.pallas_call(
        _mlp_body,
        grid=(_rows_grid(n, br),),
        in_specs=[
            pl.BlockSpec((br, D), row),
            pl.BlockSpec((br, D), row),
            pl.BlockSpec((D, 512), fix),
            pl.BlockSpec((D, 512), fix),
            pl.BlockSpec((1, 512), fix),
            pl.BlockSpec((512, 256), fix),
            pl.BlockSpec((1, 256), fix),
            pl.BlockSpec((256, 128), fix),
            pl.BlockSpec((1, 128), fix),
            pl.BlockSpec((128, 64), fix),
            pl.BlockSpec((1, 64), fix),
            pl.BlockSpec((64, 8), fix),
            pl.BlockSpec((1, 8), fix),
        ],
        out_specs=pl.BlockSpec((br, 8), row),
        out_shape=jax.ShapeDtypeStruct((n, 8), jnp.float32),
    )(z0, z1, W0[:D], W0[D:], b0.reshape(1, 512), W1, b1.reshape(1, 256),
      W2, b2.reshape(1, 128), W3, b3.reshape(1, 64), W4p, b4p.reshape(1, 8))
    return out[:, 0]


_RELS = [('before_ep', 'OER', 'OER'), ('covers', 'OER', 'Concept'),
         ('belongs', 'Concept', 'Class'), ('rev_covers', 'Concept', 'OER'),
         ('rev_belongs', 'Class', 'Concept')]


def _pad_idx(v, fill):
    e = v.shape[0]
    e_pad = _pad_to(e, _NW * _K * 2)
    if e_pad == e:
        return v
    return jnp.concatenate(
        [v, jnp.full((e_pad - e,), fill, jnp.int32)])


def kernel(x_oer, x_concept, x_class, edge_label_index_before_sr, edge_index_before_ep, edge_index_covers, edge_index_belongs, edge_index_rev_covers, edge_index_rev_belongs, lin_oer_W, lin_oer_b, lin_concept_W, lin_concept_b, lin_class_W, lin_class_b, l0_before_ep_Wl, l0_before_ep_bl, l0_before_ep_Wr, l0_covers_Wl, l0_covers_bl, l0_covers_Wr, l0_belongs_Wl, l0_belongs_bl, l0_belongs_Wr, l0_rev_covers_Wl, l0_rev_covers_bl, l0_rev_covers_Wr, l0_rev_belongs_Wl, l0_rev_belongs_bl, l0_rev_belongs_Wr, l1_before_ep_Wl, l1_before_ep_bl, l1_before_ep_Wr, l1_covers_Wl, l1_covers_bl, l1_covers_Wr, l1_belongs_Wl, l1_belongs_bl, l1_belongs_Wr, l1_rev_covers_Wl, l1_rev_covers_bl, l1_rev_covers_Wr, l1_rev_belongs_Wl, l1_rev_belongs_bl, l1_rev_belongs_Wr, mlp_W0, mlp_b0, mlp_W1, mlp_b1, mlp_W2, mlp_b2, mlp_W3, mlp_b3, mlp_W4, mlp_b4):
    params = dict(locals())
    x = {
        'OER': _proj(x_oer, lin_oer_W, lin_oer_b),
        'Concept': _proj(x_concept, lin_concept_W, lin_concept_b),
        'Class': _proj(x_class, lin_class_W, lin_class_b),
    }
    nn = {t: x[t].shape[0] for t in x}
    ndp = {t: _pad_to(nn[t] + 1, 512) for t in x}
    edges = {
        'before_ep': edge_index_before_ep,
        'covers': edge_index_covers,
        'belongs': edge_index_belongs,
        'rev_covers': edge_index_rev_covers,
        'rev_belongs': edge_index_rev_belongs,
    }
    epad = {}
    cnt = {}
    for rel, src_t, dst_t in _RELS:
        ei = edges[rel]
        src = _pad_idx(ei[0], 0)
        dst = _pad_idx(ei[1], nn[dst_t])
        epad[rel] = (src, dst)
        cnt[rel] = _sc_count(dst, ndp[dst_t])[:, :, :16]

    for l in range(2):
        acc = {'OER': [], 'Concept': [], 'Class': []}
        for rel, src_t, dst_t in _RELS:
            src, dst = epad[rel]
            s = _sc_segsum(x[src_t], src, dst, ndp[dst_t])
            acc[dst_t].append((s, cnt[rel],
                               params['l%d_%s_Wl' % (l, rel)],
                               params['l%d_%s_bl' % (l, rel)],
                               params['l%d_%s_Wr' % (l, rel)]))
        newx = {}
        for t, items in acc.items():
            newx[t] = _combine([it[0] for it in items],
                               [it[1] for it in items],
                               [it[2] for it in items],
                               [it[3] for it in items],
                               [it[4] for it in items], x[t])
        x = newx

    e = edge_label_index_before_sr
    h = x['OER']
    z0 = _sc_gather(h, _pad_idx(e[0], 0))
    z1 = _sc_gather(h, _pad_idx(e[1], 0))
    n_lab = e.shape[1]
    out = _mlp(z0, z1, mlp_W0, mlp_b0, mlp_W1, mlp_b1, mlp_W2, mlp_b2,
               mlp_W3, mlp_b3, mlp_W4, mlp_b4)
    return out[:n_lab]
